# Initial kernel scaffold; baseline (speedup 1.0000x reference)
import jax
import jax.numpy as jnp
from jax.experimental import pallas as pl


def kernel(x, W_enc, b_enc, W_dec, b_dec):
    raise NotImplementedError("write your pallas kernel here")


# speedup vs baseline: 9.7900x; 9.7900x over previous
import functools

import jax
import jax.numpy as jnp
from jax.experimental import pallas as pl
from jax.experimental.pallas import tpu as pltpu

K_TOP = 64


def _encode_kernel(x_ref, w_ref, b_ref, out_ref):
    out_ref[...] = (
        jnp.dot(x_ref[...], w_ref[...], preferred_element_type=jnp.float32)
        + b_ref[...]
    )


def _float_to_key(s):
    return jnp.where(s >= 0, s, jnp.bitwise_xor(jnp.bitwise_not(s), jnp.int32(-2147483648)))


def _key_to_float_bits(t):
    return jnp.where(
        t >= 0, t, jnp.bitwise_not(jnp.bitwise_xor(t, jnp.int32(-2147483648)))
    )


def _threshold_kernel(p_ref, thr_ref, *, k):
    p = p_ref[...]
    s = jax.lax.bitcast_convert_type(p, jnp.int32)
    key = _float_to_key(s)
    rows = p.shape[0]
    t = jnp.full((rows, 1), jnp.int32(-2147483648))
    for b in range(31, -1, -1):
        t_try = jnp.zeros_like(t) if b == 31 else t + jnp.int32(1 << b)
        cnt = jnp.sum((key >= t_try).astype(jnp.float32), axis=1, keepdims=True)
        t = jnp.where(cnt >= float(k), t_try, t)
    bits = _key_to_float_bits(t)
    thr_ref[...] = jax.lax.bitcast_convert_type(bits, jnp.float32)


def _decode_kernel(p_ref, thr_ref, w_ref, b_ref, lat_ref, out_ref):
    lt = pl.program_id(1)
    p = p_ref[...]
    lat = jnp.where(p >= thr_ref[...], jnp.maximum(p, 0.0), 0.0)
    lat_ref[...] = lat
    contrib = jnp.dot(lat, w_ref[...], preferred_element_type=jnp.float32)

    @pl.when(lt == 0)
    def _():
        out_ref[...] = contrib + b_ref[...]

    @pl.when(lt != 0)
    def _():
        out_ref[...] += contrib


@jax.jit
def kernel(x, W_enc, b_enc, W_dec, b_dec):
    n, d = x.shape
    l = W_enc.shape[1]

    r1 = min(512, n)
    lt_size = min(2048, l)
    n_lt = l // lt_size
    n_nb = n // r1

    b_enc2 = b_enc.reshape(1, l)
    b_dec2 = b_dec.reshape(1, d)

    preact = pl.pallas_call(
        _encode_kernel,
        grid=(n_lt, n_nb),
        in_specs=[
            pl.BlockSpec((r1, d), lambda lt, nb: (nb, 0)),
            pl.BlockSpec((d, lt_size), lambda lt, nb: (0, lt)),
            pl.BlockSpec((1, lt_size), lambda lt, nb: (0, lt)),
        ],
        out_specs=pl.BlockSpec((r1, lt_size), lambda lt, nb: (nb, lt)),
        out_shape=jax.ShapeDtypeStruct((n, l), jnp.float32),
        compiler_params=pltpu.CompilerParams(
            dimension_semantics=("arbitrary", "arbitrary"),
        ),
    )(x, W_enc, b_enc2)

    r_thr = min(128, n)
    thresholds = pl.pallas_call(
        functools.partial(_threshold_kernel, k=K_TOP),
        grid=(n // r_thr,),
        in_specs=[pl.BlockSpec((r_thr, l), lambda i: (i, 0))],
        out_specs=pl.BlockSpec((r_thr, 1), lambda i: (i, 0)),
        out_shape=jax.ShapeDtypeStruct((n, 1), jnp.float32),
    )(preact)

    r2 = min(1024, n)
    latents, out = pl.pallas_call(
        _decode_kernel,
        grid=(n // r2, n_lt),
        in_specs=[
            pl.BlockSpec((r2, lt_size), lambda nb, lt: (nb, lt)),
            pl.BlockSpec((r2, 1), lambda nb, lt: (nb, 0)),
            pl.BlockSpec((lt_size, d), lambda nb, lt: (lt, 0)),
            pl.BlockSpec((1, d), lambda nb, lt: (0, 0)),
        ],
        out_specs=[
            pl.BlockSpec((r2, lt_size), lambda nb, lt: (nb, lt)),
            pl.BlockSpec((r2, d), lambda nb, lt: (nb, 0)),
        ],
        out_shape=[
            jax.ShapeDtypeStruct((n, l), jnp.float32),
            jax.ShapeDtypeStruct((n, d), jnp.float32),
        ],
        compiler_params=pltpu.CompilerParams(
            dimension_semantics=("parallel", "arbitrary"),
        ),
    )(preact, thresholds, W_dec, b_dec2)

    num_dead = jnp.array(0, dtype=jnp.int32)
    return (latents, out, preact, num_dead)
